# baseline (speedup 1.0000x reference)
import functools

import jax
import jax.numpy as jnp
from jax import lax
from jax.experimental import pallas as pl
from jax.experimental.pallas import tpu as pltpu
from jax.experimental.pallas import tpu_sc as plsc

B = 16384
K = 26
EMB = 64
BK = B * K

NC = 2
NS = 16
NW = NC * NS
ROWS_PER_W = BK // NW
CHUNK = 832
N_CHUNKS = ROWS_PER_W // CHUNK


P = 8
ROWS_P = BK // P


def _make_gather(n_rows):
    n2 = n_rows // 2
    rows_per_w = n2 // NW
    n_chunks = rows_per_w // CHUNK
    mesh = plsc.VectorSubcoreMesh(core_axis_name="c", subcore_axis_name="s")

    @functools.partial(
        pl.kernel,
        out_type=jax.ShapeDtypeStruct((n2, 2, EMB), jnp.float32),
        mesh=mesh,
        scratch_types=[
            pltpu.VMEM((CHUNK,), jnp.int32),
            pltpu.VMEM((CHUNK, EMB), jnp.float32),
            pltpu.SemaphoreType.DMA,
        ],
        compiler_params=pltpu.CompilerParams(use_tc_tiling_on_sc=False),
    )
    def gather_k(idx_hbm, table_hbm, out_hbm, idx_v, rows_v, sem):
        wid = lax.axis_index("s") * NC + lax.axis_index("c")
        base = wid * rows_per_w

        def body(i, carry):
            off = base + i * CHUNK
            pltpu.sync_copy(idx_hbm.at[pl.ds(off, CHUNK)], idx_v)
            pltpu.async_copy(table_hbm.at[idx_v], rows_v, sem).wait()
            pltpu.sync_copy(rows_v, out_hbm.at[pl.ds(off, CHUNK), 0])
            pltpu.sync_copy(idx_hbm.at[pl.ds(n2 + off, CHUNK)], idx_v)
            pltpu.async_copy(table_hbm.at[idx_v], rows_v, sem).wait()
            pltpu.sync_copy(rows_v, out_hbm.at[pl.ds(off, CHUNK), 1])
            return carry

        lax.fori_loop(0, n_chunks, body, 0, unroll=False)

    return gather_k


_gather_piece = _make_gather(ROWS_P)

TOTAL = 1040000
CB = 3200


def _transpose_body(xt_ref, o_ref):
    xt = xt_ref[...]
    eye = (
        lax.broadcasted_iota(jnp.int32, (EMB, EMB), 0)
        == lax.broadcasted_iota(jnp.int32, (EMB, EMB), 1)
    ).astype(jnp.float32)
    y = lax.dot_general(
        xt, eye, (((0,), (0,)), ((), ())), preferred_element_type=jnp.float32
    )
    o_ref[:, :EMB] = y[: CB // 2]
    o_ref[:, EMB:] = y[CB // 2 :]


def _transpose_table(tableT):
    grid = (TOTAL // CB,)
    return pl.pallas_call(
        _transpose_body,
        out_shape=jax.ShapeDtypeStruct((TOTAL // 2, 2 * EMB), jnp.float32),
        grid=grid,
        in_specs=[pl.BlockSpec((EMB, CB), lambda i: (0, i))],
        out_specs=pl.BlockSpec((CB // 2, 2 * EMB), lambda i: (i, 0)),
    )(tableT)


BL = 2048


def _erf(x):
    s = jnp.sign(x)
    a = jnp.abs(x)
    t = 1.0 / (1.0 + 0.3275911 * a)
    poly = t * (
        0.254829592
        + t * (-0.284496736 + t * (1.421413741 + t * (-1.453152027 + t * 1.061405429)))
    )
    return s * (1.0 - poly * jnp.exp(-a * a))


def _gelu_exact(x):
    return 0.5 * x * (1.0 + _erf(x * 0.7071067811865476))


def _mlp_t_half(x, w1, b1, w2, b2):
    h = lax.dot_general(
        w1, x, (((0,), (1,)), ((), ())), preferred_element_type=jnp.float32
    ) + b1
    h = _gelu_exact(h)
    return lax.dot_general(
        w2, h, (((0,), (0,)), ((), ())), preferred_element_type=jnp.float32
    ) + b2


def _mlp_t_body(x_ref, w1_ref, b1_ref, w2_ref, b2_ref, o_ref):
    h = pl.program_id(1)
    x2 = x_ref[...]
    x = jnp.where(h == 0, x2[:, :EMB], x2[:, EMB:])
    o_ref[0] = _mlp_t_half(x, w1_ref[...], b1_ref[...], w2_ref[...], b2_ref[...])


def _mlp_t_first_body(x_ref, w1_ref, b1_ref, w2_ref, b2_ref, o_ref):
    _mlp_t_body(x_ref, w1_ref, b1_ref, w2_ref, b2_ref, o_ref)


def _mlp_t_acc_body(x_ref, w1_ref, b1_ref, w2_ref, b2_ref, _prev_ref, o_ref):
    _mlp_t_body(x_ref, w1_ref, b1_ref, w2_ref, b2_ref, o_ref)


HB = 1024
NBH = B // HB
S2 = ROWS_P // 2 // HB


def _mlp_t_piece(piece_idx, emb2_p, W1, b1, W2, b2, prev_out):
    off = piece_idx * 2 * S2
    common_specs = [
        pl.BlockSpec((HB, 2 * EMB), lambda s, h: (s, 0)),
        pl.BlockSpec((EMB, EMB), lambda s, h: (0, 0)),
        pl.BlockSpec((EMB, 1), lambda s, h: (0, 0)),
        pl.BlockSpec((EMB, EMB), lambda s, h: (0, 0)),
        pl.BlockSpec((EMB, 1), lambda s, h: (0, 0)),
    ]
    out_spec = pl.BlockSpec(
        (1, EMB, HB),
        lambda s, h: (
            (off + h * S2 + s) // NBH,
            0,
            (off + h * S2 + s) % NBH,
        ),
    )
    args = (emb2_p, W1, b1.reshape(EMB, 1), W2, b2.reshape(EMB, 1))
    if piece_idx == 0:
        return pl.pallas_call(
            _mlp_t_first_body,
            out_shape=jax.ShapeDtypeStruct((K, EMB, B), jnp.float32),
            grid=(S2, 2),
            in_specs=common_specs,
            out_specs=out_spec,
        )(*args)
    return pl.pallas_call(
        _mlp_t_acc_body,
        out_shape=jax.ShapeDtypeStruct((K, EMB, B), jnp.float32),
        grid=(S2, 2),
        in_specs=common_specs + [pl.BlockSpec(memory_space=pl.ANY)],
        out_specs=out_spec,
        input_output_aliases={5: 0},
    )(*args, prev_out)


def kernel(attrs, table, shift, W1, b1, W2, b2):
    idx = (attrs.astype(jnp.int32) + shift.astype(jnp.int32)).T.reshape(BK)
    j = idx % CB
    idxp = idx - j + 2 * (j % (CB // 2)) + j // (CB // 2)
    table_lin = _transpose_table(table.T).reshape(TOTAL, EMB)
    out_t = None
    for p in range(P):
        idx_p = lax.slice(idxp, (p * ROWS_P,), ((p + 1) * ROWS_P,))
        emb_p = _gather_piece(idx_p, table_lin)
        emb2_p = emb_p.reshape(ROWS_P // 2, 2 * EMB)
        out_t = _mlp_t_piece(p, emb2_p, W1, b1, W2, b2, out_t)
    return jnp.transpose(out_t, (2, 0, 1))


# speedup vs baseline: 1.7570x; 1.7570x over previous
import functools

import jax
import jax.numpy as jnp
from jax import lax
from jax.experimental import pallas as pl
from jax.experimental.pallas import tpu as pltpu
from jax.experimental.pallas import tpu_sc as plsc

B = 16384
K = 26
EMB = 64
BK = B * K

NC = 2
NS = 16
NW = NC * NS
CHUNK = 832

P = 8
ROWS_P = BK // P


def _make_gather(n_rows):
    rows_per_w = n_rows // NW
    n_chunks = rows_per_w // CHUNK
    mesh = plsc.VectorSubcoreMesh(core_axis_name="c", subcore_axis_name="s")

    @functools.partial(
        pl.kernel,
        out_type=jax.ShapeDtypeStruct((n_rows, EMB), jnp.float32),
        mesh=mesh,
        scratch_types=[
            pltpu.VMEM((CHUNK,), jnp.int32),
            pltpu.VMEM((CHUNK, EMB), jnp.float32),
            pltpu.SemaphoreType.DMA,
        ],
        compiler_params=pltpu.CompilerParams(use_tc_tiling_on_sc=False),
    )
    def gather_k(idx_hbm, table_hbm, out_hbm, idx_v, rows_v, sem):
        wid = lax.axis_index("s") * NC + lax.axis_index("c")
        base = wid * rows_per_w

        def body(i, carry):
            off = base + i * CHUNK
            pltpu.sync_copy(idx_hbm.at[pl.ds(off, CHUNK)], idx_v)
            pltpu.async_copy(table_hbm.at[idx_v], rows_v, sem).wait()
            pltpu.sync_copy(rows_v, out_hbm.at[pl.ds(off, CHUNK)])
            return carry

        lax.fori_loop(0, n_chunks, body, 0, unroll=False)

    return gather_k


_gather_piece = _make_gather(ROWS_P)

TOTAL = 1040000
CB = 3200


def _transpose_body(xt_ref, o_ref):
    xt = xt_ref[...]
    eye = (
        lax.broadcasted_iota(jnp.int32, (EMB, EMB), 0)
        == lax.broadcasted_iota(jnp.int32, (EMB, EMB), 1)
    ).astype(jnp.float32)
    y = lax.dot_general(
        xt, eye, (((0,), (0,)), ((), ())), preferred_element_type=jnp.float32
    )
    o_ref[:, :EMB] = y[: CB // 2]
    o_ref[:, EMB:] = y[CB // 2 :]


def _transpose_table(tableT):
    grid = (TOTAL // CB,)
    return pl.pallas_call(
        _transpose_body,
        out_shape=jax.ShapeDtypeStruct((TOTAL // 2, 2 * EMB), jnp.float32),
        grid=grid,
        in_specs=[pl.BlockSpec((EMB, CB), lambda i: (0, i))],
        out_specs=pl.BlockSpec((CB // 2, 2 * EMB), lambda i: (i, 0)),
    )(tableT)


BL = 2048


def _erf(x):
    s = jnp.sign(x)
    a = jnp.abs(x)
    t = 1.0 / (1.0 + 0.3275911 * a)
    poly = t * (
        0.254829592
        + t * (-0.284496736 + t * (1.421413741 + t * (-1.453152027 + t * 1.061405429)))
    )
    return s * (1.0 - poly * jnp.exp(-a * a))


def _gelu_exact(x):
    return 0.5 * x * (1.0 + _erf(x * 0.7071067811865476))


def _mlp_t_half(x, w1, b1, w2, b2):
    h = lax.dot_general(
        w1, x, (((0,), (1,)), ((), ())), preferred_element_type=jnp.float32
    ) + b1
    h = _gelu_exact(h)
    return lax.dot_general(
        w2, h, (((0,), (0,)), ((), ())), preferred_element_type=jnp.float32
    ) + b2


def _mlp_t_body(x_ref, w1_ref, b1_ref, w2_ref, b2_ref, o_ref):
    x2 = x_ref[...]
    w1, b1 = w1_ref[...], b1_ref[...]
    w2, b2 = w2_ref[...], b2_ref[...]
    o_ref[0, :, : BL // 2] = _mlp_t_half(x2[:, :EMB], w1, b1, w2, b2)
    o_ref[0, :, BL // 2 :] = _mlp_t_half(x2[:, EMB:], w1, b1, w2, b2)


def _mlp_t_first_body(x_ref, w1_ref, b1_ref, w2_ref, b2_ref, o_ref):
    _mlp_t_body(x_ref, w1_ref, b1_ref, w2_ref, b2_ref, o_ref)


def _mlp_t_acc_body(x_ref, w1_ref, b1_ref, w2_ref, b2_ref, _prev_ref, o_ref):
    _mlp_t_body(x_ref, w1_ref, b1_ref, w2_ref, b2_ref, o_ref)


NB = B // BL
STEPS_P = (K * NB) // P


def _mlp_t_piece(piece_idx, emb2_p, W1, b1, W2, b2, prev_out):
    off = piece_idx * STEPS_P
    common_specs = [
        pl.BlockSpec((BL // 2, 2 * EMB), lambda s: (s, 0)),
        pl.BlockSpec((EMB, EMB), lambda s: (0, 0)),
        pl.BlockSpec((EMB, 1), lambda s: (0, 0)),
        pl.BlockSpec((EMB, EMB), lambda s: (0, 0)),
        pl.BlockSpec((EMB, 1), lambda s: (0, 0)),
    ]
    out_spec = pl.BlockSpec(
        (1, EMB, BL), lambda s: ((off + s) // NB, 0, (off + s) % NB)
    )
    args = (emb2_p, W1, b1.reshape(EMB, 1), W2, b2.reshape(EMB, 1))
    if piece_idx == 0:
        return pl.pallas_call(
            _mlp_t_first_body,
            out_shape=jax.ShapeDtypeStruct((K, EMB, B), jnp.float32),
            grid=(STEPS_P,),
            in_specs=common_specs,
            out_specs=out_spec,
        )(*args)
    return pl.pallas_call(
        _mlp_t_acc_body,
        out_shape=jax.ShapeDtypeStruct((K, EMB, B), jnp.float32),
        grid=(STEPS_P,),
        in_specs=common_specs + [pl.BlockSpec(memory_space=pl.ANY)],
        out_specs=out_spec,
        input_output_aliases={5: 0},
    )(*args, prev_out)


def kernel(attrs, table, shift, W1, b1, W2, b2):
    idx = (attrs.astype(jnp.int32) + shift.astype(jnp.int32)).T.reshape(BK)
    j = idx % CB
    idxp = idx - j + 2 * (j % (CB // 2)) + j // (CB // 2)
    idxs = idxp.reshape(BK // BL, 2, BL // 2).swapaxes(1, 2).reshape(BK)
    table_lin = _transpose_table(table.T).reshape(TOTAL, EMB)
    out_t = None
    for p in range(P):
        idx_p = lax.slice(idxs, (p * ROWS_P,), ((p + 1) * ROWS_P,))
        emb_p = _gather_piece(idx_p, table_lin)
        emb2_p = emb_p.reshape(ROWS_P // 2, 2 * EMB)
        out_t = _mlp_t_piece(p, emb2_p, W1, b1, W2, b2, out_t)
    return jnp.transpose(out_t, (2, 0, 1))
